# baseline (speedup 1.0000x reference)
import jax
import jax.numpy as jnp
from jax import lax
from jax.experimental import pallas as pl
from jax.experimental.pallas import tpu as pltpu

_N = 20000
_NPAD = 20480
_ROWS = 160
_COLS = 128
_K = 5000
_TOPK = 750
_IOU_T = 0.3
_CONF_T = 0.05
_VAR0 = 0.1
_VAR1 = 0.2
_OUT_ROWS = 768


def _body(sc_ref, lx_ref, ly_ref, lw_ref, lh_ref,
          px_ref, py_ref, pw_ref, ph_ref, out_ref,
          x1_ref, y1_ref, x2_ref, y2_ref, ar_ref, gi_ref, ms_ref):
    f32 = jnp.float32
    i32 = jnp.int32
    score = sc_ref[...]
    pw = pw_ref[...]
    ph = ph_ref[...]

    cx = px_ref[...] + lx_ref[...] * f32(_VAR0) * pw
    cy = py_ref[...] + ly_ref[...] * f32(_VAR0) * ph
    w = pw * jnp.exp(lw_ref[...] * f32(_VAR1))
    h = ph * jnp.exp(lh_ref[...] * f32(_VAR1))
    x1 = cx - w / f32(2.0)
    y1 = cy - h / f32(2.0)
    x2 = x1 + w
    y2 = y1 + h
    x1_ref[...] = x1
    y1_ref[...] = y1
    x2_ref[...] = x2
    y2_ref[...] = y2
    ar_ref[...] = (x2 - x1) * (y2 - y1)
    out_ref[...] = jnp.zeros((_OUT_ROWS, _COLS), f32)

    gidx = (lax.broadcasted_iota(i32, (_ROWS, _COLS), 0) * _COLS
            + lax.broadcasted_iota(i32, (_ROWS, _COLS), 1))
    gi_ref[...] = gidx

    valid = score > f32(_CONF_T)
    key = jnp.where(valid, lax.bitcast_convert_type(score, i32), i32(-1))

    def _bs_val(_, lohi):
        lo, hi = lohi
        mid = lo + (hi - lo) // 2
        c = jnp.sum((key >= mid).astype(i32))
        take = c >= _K
        return (jnp.where(take, mid, lo), jnp.where(take, hi, mid))

    lo, _ = lax.fori_loop(0, 31, _bs_val, (i32(-1), i32(0x7F800000)))
    sstar = lo
    cgt = jnp.sum((key > sstar).astype(i32))
    need = i32(_K) - cgt
    tie = key == sstar

    def _bs_idx(_, lohi):
        lo, hi = lohi
        mid = lo + (hi - lo) // 2
        c = jnp.sum((tie & (gidx >= mid)).astype(i32))
        take = c >= need
        return (jnp.where(take, mid, lo), jnp.where(take, hi, mid))

    lo2, _ = lax.fori_loop(0, 15, _bs_idx, (i32(0), i32(_NPAD)))
    in_top = (key > sstar) | (tie & (gidx >= lo2))

    neg = f32(-jnp.inf)
    msc0 = jnp.where(valid & in_top, score, neg)
    ms_ref[...] = msc0

    lane = lax.broadcasted_iota(i32, (1, _COLS), 1)
    zero = f32(0.0)
    iou_t = f32(_IOU_T)

    ones_m = jnp.ones((_COLS, _COLS), f32)
    _DOT = dict(dimension_numbers=(((1,), (0,)), ((), ())),
                preferred_element_type=f32,
                precision=jax.lax.Precision.HIGHEST)

    def _extract(oh, planes):
        rows = jnp.concatenate(
            [jnp.sum(jnp.where(oh, p, zero), axis=0).reshape(1, _COLS)
             for p in planes[:4]], axis=0)
        rep = lax.dot_general(rows, ones_m, **_DOT)
        x1p, y1p, x2p, y2p = (rep[0:1, :], rep[1:2, :],
                              rep[2:3, :], rep[3:4, :])
        return x1p, y1p, x2p, y2p, (x2p - x1p) * (y2p - y1p)

    def _keep_plane(b, planes):
        x1p, y1p, x2p, y2p, areap = b
        x1a, y1a, x2a, y2a, ara = planes
        iw = jnp.maximum(jnp.minimum(x2a, x2p) - jnp.maximum(x1a, x1p), zero)
        ih = jnp.maximum(jnp.minimum(y2a, y2p) - jnp.maximum(y1a, y1p), zero)
        inter = iw * ih
        union = ara - inter + areap
        return (inter / union) <= iou_t

    def _row(mx, b):
        x1p, y1p, x2p, y2p, _ = b
        return jnp.where(lane == 0, mx,
               jnp.where(lane == 1, x1p,
               jnp.where(lane == 2, y1p,
               jnp.where(lane == 3, x2p,
               jnp.where(lane == 4, y2p, zero)))))

    def _cond(state):
        t, mx1 = state
        return (t < _TOPK) & (mx1 > neg)

    def _pick(state):
        t, mx1 = state
        msc = ms_ref[...]
        gi = gi_ref[...]
        planes = (x1_ref[...], y1_ref[...], x2_ref[...], y2_ref[...],
                  ar_ref[...])
        eq1 = msc == mx1
        pos1 = jnp.max(jnp.where(eq1, gi, i32(-1)))
        mx2c = jnp.max(jnp.where(eq1, neg, msc))
        cnt1 = jnp.sum(eq1.astype(f32), axis=0).reshape(1, _COLS)
        nmxv = lax.dot_general(cnt1, ones_m, **_DOT)
        mx2 = jnp.where(nmxv >= f32(2.0), mx1, mx2c)
        pos2 = jnp.max(jnp.where((msc == mx2)
                                 & ((mx2 != mx1) | (gi < pos1)),
                                 gi, i32(-1)))
        b1 = _extract(gi == pos1, planes)
        b2 = _extract(gi == pos2, planes)

        iw = jnp.maximum(jnp.minimum(b2[2], b1[2])
                         - jnp.maximum(b2[0], b1[0]), zero)
        ih = jnp.maximum(jnp.minimum(b2[3], b1[3])
                         - jnp.maximum(b2[1], b1[1]), zero)
        inter12 = iw * ih
        iou12 = inter12 / (b2[4] - inter12 + b1[4])
        commit2 = (mx2 > neg) & (iou12 <= iou_t)

        keep = (_keep_plane(b1, planes)
                & (_keep_plane(b2, planes) | jnp.logical_not(commit2)))
        msc = jnp.where(keep, msc, neg)
        ms_ref[...] = msc

        c2s = commit2.astype(i32)[0, 0]
        out_ref[pl.ds(t, 1), :] = _row(mx1, b1)
        t2 = jnp.where(c2s == 1, t + 1, i32(_TOPK + 1))
        out_ref[pl.ds(t2, 1), :] = _row(mx2, b2)

        return t + 1 + c2s, jnp.max(msc)

    lax.while_loop(_cond, _pick, (i32(0), jnp.max(msc0)))


_SCRATCH = [pltpu.VMEM((_ROWS, _COLS), jnp.float32)] * 5 \
           + [pltpu.VMEM((_ROWS, _COLS), jnp.int32),
              pltpu.VMEM((_ROWS, _COLS), jnp.float32)]


def kernel(loc_data, conf_data, prior_data):
    num = loc_data.shape[0]
    f32 = jnp.float32

    def plane(a):
        return jnp.pad(a.astype(f32), (0, _NPAD - _N)).reshape(_ROWS, _COLS)

    scores = conf_data[0, :, 1]
    loc = loc_data[0]
    args = [plane(scores),
            plane(loc[:, 0]), plane(loc[:, 1]),
            plane(loc[:, 2]), plane(loc[:, 3]),
            plane(prior_data[:, 0]), plane(prior_data[:, 1]),
            plane(prior_data[:, 2]), plane(prior_data[:, 3])]

    res = pl.pallas_call(
        _body,
        out_shape=jax.ShapeDtypeStruct((_OUT_ROWS, _COLS), f32),
        scratch_shapes=_SCRATCH,
    )(*args)

    out = jnp.zeros((num, 2, _TOPK, 5), dtype=f32)
    return out.at[0, 1].set(res[:_TOPK, :5])


# speedup vs baseline: 1.1211x; 1.1211x over previous
import jax
import jax.numpy as jnp
from jax import lax
from jax.experimental import pallas as pl
from jax.experimental.pallas import tpu as pltpu

_N = 20000
_NPAD = 20480
_ROWS = 160
_COLS = 128
_K = 5000
_TOPK = 750
_IOU_T = 0.3
_CONF_T = 0.05
_VAR0 = 0.1
_VAR1 = 0.2
_OUT_ROWS = 768


def _body(sc_ref, lx_ref, ly_ref, lw_ref, lh_ref,
          px_ref, py_ref, pw_ref, ph_ref, out_ref,
          x1_ref, y1_ref, x2_ref, y2_ref, ar_ref, gi_ref, ms_ref):
    f32 = jnp.float32
    i32 = jnp.int32
    score = sc_ref[...]
    pw = pw_ref[...]
    ph = ph_ref[...]

    cx = px_ref[...] + lx_ref[...] * f32(_VAR0) * pw
    cy = py_ref[...] + ly_ref[...] * f32(_VAR0) * ph
    w = pw * jnp.exp(lw_ref[...] * f32(_VAR1))
    h = ph * jnp.exp(lh_ref[...] * f32(_VAR1))
    x1 = cx - w / f32(2.0)
    y1 = cy - h / f32(2.0)
    x2 = x1 + w
    y2 = y1 + h
    x1_ref[...] = x1
    y1_ref[...] = y1
    x2_ref[...] = x2
    y2_ref[...] = y2
    ar_ref[...] = (x2 - x1) * (y2 - y1)
    out_ref[...] = jnp.zeros((_OUT_ROWS, _COLS), f32)

    gidx = (lax.broadcasted_iota(i32, (_ROWS, _COLS), 0) * _COLS
            + lax.broadcasted_iota(i32, (_ROWS, _COLS), 1))
    gi_ref[...] = gidx

    valid = score > f32(_CONF_T)
    key = jnp.where(valid, lax.bitcast_convert_type(score, i32), i32(-1))

    def _bs_val(_, lohi):
        lo, hi = lohi
        mid = lo + (hi - lo) // 2
        c = jnp.sum((key >= mid).astype(i32))
        take = c >= _K
        return (jnp.where(take, mid, lo), jnp.where(take, hi, mid))

    lo, _ = lax.fori_loop(0, 31, _bs_val, (i32(-1), i32(0x7F800000)))
    sstar = lo
    cgt = jnp.sum((key > sstar).astype(i32))
    need = i32(_K) - cgt
    tie = key == sstar

    def _bs_idx(_, lohi):
        lo, hi = lohi
        mid = lo + (hi - lo) // 2
        c = jnp.sum((tie & (gidx >= mid)).astype(i32))
        take = c >= need
        return (jnp.where(take, mid, lo), jnp.where(take, hi, mid))

    lo2, _ = lax.fori_loop(0, 15, _bs_idx, (i32(0), i32(_NPAD)))
    in_top = (key > sstar) | (tie & (gidx >= lo2))

    neg = f32(-jnp.inf)
    msc0 = jnp.where(valid & in_top, score, neg)
    ms_ref[...] = msc0

    lane = lax.broadcasted_iota(i32, (1, _COLS), 1)
    zero = f32(0.0)
    iou_t = f32(_IOU_T)

    def _extract(pos):
        r = pos // _COLS
        c = pos - r * _COLS
        loh = lane == c
        x1p = jnp.sum(jnp.where(loh, x1_ref[pl.ds(r, 1), :], zero))
        y1p = jnp.sum(jnp.where(loh, y1_ref[pl.ds(r, 1), :], zero))
        x2p = jnp.sum(jnp.where(loh, x2_ref[pl.ds(r, 1), :], zero))
        y2p = jnp.sum(jnp.where(loh, y2_ref[pl.ds(r, 1), :], zero))
        return x1p, y1p, x2p, y2p, (x2p - x1p) * (y2p - y1p)

    def _keep_plane(b, planes):
        x1p, y1p, x2p, y2p, areap = b
        x1a, y1a, x2a, y2a, ara = planes
        iw = jnp.maximum(jnp.minimum(x2a, x2p) - jnp.maximum(x1a, x1p), zero)
        ih = jnp.maximum(jnp.minimum(y2a, y2p) - jnp.maximum(y1a, y1p), zero)
        inter = iw * ih
        union = ara - inter + areap
        return (inter / union) <= iou_t

    def _row(mx, b):
        x1p, y1p, x2p, y2p, _ = b
        return jnp.where(lane == 0, mx,
               jnp.where(lane == 1, x1p,
               jnp.where(lane == 2, y1p,
               jnp.where(lane == 3, x2p,
               jnp.where(lane == 4, y2p, zero)))))

    def _cond(state):
        t, mx1 = state
        return (t < _TOPK) & (mx1 > neg)

    def _pick(state):
        t, mx1 = state
        msc = ms_ref[...]
        gi = gi_ref[...]
        planes = (x1_ref[...], y1_ref[...], x2_ref[...], y2_ref[...],
                  ar_ref[...])
        eq1 = msc == mx1
        pos1 = jnp.max(jnp.where(eq1, gi, i32(-1)))
        nmx = jnp.sum(eq1.astype(i32))
        mx2c = jnp.max(jnp.where(eq1, neg, msc))
        mx2 = jnp.where(nmx >= 2, mx1, mx2c)
        pos2 = jnp.max(jnp.where((msc == mx2)
                                 & ((mx2 != mx1) | (gi < pos1)),
                                 gi, i32(-1)))
        b1 = _extract(pos1)
        b2 = _extract(pos2)

        iw = jnp.maximum(jnp.minimum(b2[2], b1[2])
                         - jnp.maximum(b2[0], b1[0]), zero)
        ih = jnp.maximum(jnp.minimum(b2[3], b1[3])
                         - jnp.maximum(b2[1], b1[1]), zero)
        inter12 = iw * ih
        iou12 = inter12 / (b2[4] - inter12 + b1[4])
        commit2 = (mx2 > neg) & (iou12 <= iou_t)

        keep = (_keep_plane(b1, planes)
                & (_keep_plane(b2, planes) | jnp.logical_not(commit2)))
        msc = jnp.where(keep, msc, neg)
        ms_ref[...] = msc

        out_ref[pl.ds(t, 1), :] = _row(mx1, b1)
        t2 = jnp.where(commit2, t + 1, i32(_TOPK + 1))
        out_ref[pl.ds(t2, 1), :] = _row(mx2, b2)

        return t + 1 + commit2.astype(i32), jnp.max(msc)

    lax.while_loop(_cond, _pick, (i32(0), jnp.max(msc0)))


_SCRATCH = [pltpu.VMEM((_ROWS, _COLS), jnp.float32)] * 5 \
           + [pltpu.VMEM((_ROWS, _COLS), jnp.int32),
              pltpu.VMEM((_ROWS, _COLS), jnp.float32)]


def kernel(loc_data, conf_data, prior_data):
    num = loc_data.shape[0]
    f32 = jnp.float32

    def plane(a):
        return jnp.pad(a.astype(f32), (0, _NPAD - _N)).reshape(_ROWS, _COLS)

    scores = conf_data[0, :, 1]
    loc = loc_data[0]
    args = [plane(scores),
            plane(loc[:, 0]), plane(loc[:, 1]),
            plane(loc[:, 2]), plane(loc[:, 3]),
            plane(prior_data[:, 0]), plane(prior_data[:, 1]),
            plane(prior_data[:, 2]), plane(prior_data[:, 3])]

    res = pl.pallas_call(
        _body,
        out_shape=jax.ShapeDtypeStruct((_OUT_ROWS, _COLS), f32),
        scratch_shapes=_SCRATCH,
    )(*args)

    out = jnp.zeros((num, 2, _TOPK, 5), dtype=f32)
    return out.at[0, 1].set(res[:_TOPK, :5])
